# baseline (speedup 1.0000x reference)
import jax
import jax.numpy as jnp
from jax.experimental import pallas as pl
from jax.experimental.pallas import tpu as pltpu

N = 4096
D = 512
BLK = 4096
GRID = N // BLK


def _body(sidx_ref, adj_ref, uni_ref, sub_ref, orig_ref, out_ref):
    g = pl.program_id(0)
    q = sidx_ref[0]

    @pl.when(g == 0)
    def _init():
        out_ref[...] = jnp.zeros((8, D), jnp.float32)
        out_ref[0:1, :] = orig_ref[...]

    lane = q % 128
    lane_ids = jax.lax.broadcasted_iota(jnp.int32, (BLK, 128), 1)
    colvals = jnp.sum(jnp.where(lane_ids == lane, adj_ref[...], 0.0), axis=1,
                      keepdims=True)
    maskf = (colvals > 0.0).astype(jnp.float32)

    u = uni_ref[...]
    s = sub_ref[...]
    out_ref[2:3, :] += jnp.sum(u * maskf, axis=0, keepdims=True)
    out_ref[3:4, :] += jnp.sum(u, axis=0, keepdims=True)
    out_ref[4:5, :] += jnp.sum(s, axis=0, keepdims=True)

    @pl.when(g == q // BLK)
    def _cur():
        local = q - g * BLK
        row_ids = jax.lax.broadcasted_iota(jnp.int32, (BLK, D), 0)
        out_ref[1:2, :] = jnp.sum(jnp.where(row_ids == local, s, 0.0), axis=0,
                                  keepdims=True)


def kernel(adj, cur_sub_idx, uni_feat, sub_feat, original_sub_feat):
    sidx = jnp.asarray(cur_sub_idx, jnp.int32).reshape((1,))
    orig = original_sub_feat.reshape((1, D))
    grid_spec = pltpu.PrefetchScalarGridSpec(
        num_scalar_prefetch=1,
        grid=(GRID,),
        in_specs=[
            pl.BlockSpec((BLK, 128), lambda g, s: (g, s[0] // 128)),
            pl.BlockSpec((BLK, D), lambda g, s: (g, 0)),
            pl.BlockSpec((BLK, D), lambda g, s: (g, 0)),
            pl.BlockSpec((1, D), lambda g, s: (0, 0)),
        ],
        out_specs=pl.BlockSpec((8, D), lambda g, s: (0, 0)),
    )
    out = pl.pallas_call(
        _body,
        grid_spec=grid_spec,
        out_shape=jax.ShapeDtypeStruct((8, D), jnp.float32),
    )(sidx, adj, uni_feat, sub_feat, orig)
    return out[:5].reshape(-1)


# speedup vs baseline: 1.1669x; 1.1669x over previous
import jax
import jax.numpy as jnp
from jax.experimental import pallas as pl
from jax.experimental.pallas import tpu as pltpu

N = 4096
D = 512
BLK = 2048
GRID = N // BLK


def _body(sidx_ref, adj_ref, uni_ref, sub_ref, subrow_ref, orig_ref, out_ref):
    g = pl.program_id(0)
    q = sidx_ref[0]

    @pl.when(g == 0)
    def _init():
        out_ref[...] = jnp.zeros((8, D), jnp.float32)
        out_ref[0:1, :] = orig_ref[...]
        local = q % 8
        row_ids = jax.lax.broadcasted_iota(jnp.int32, (8, D), 0)
        out_ref[1:2, :] = jnp.sum(
            jnp.where(row_ids == local, subrow_ref[...], 0.0), axis=0,
            keepdims=True)

    lane = q % 128
    lane_ids = jax.lax.broadcasted_iota(jnp.int32, (BLK, 128), 1)
    colvals = jnp.sum(jnp.where(lane_ids == lane, adj_ref[...], 0.0), axis=1,
                      keepdims=True)
    maskf = (colvals > 0.0).astype(jnp.float32)

    u = uni_ref[...]
    s = sub_ref[...]
    out_ref[2:3, :] += jnp.sum(u * maskf, axis=0, keepdims=True)
    out_ref[3:4, :] += jnp.sum(u, axis=0, keepdims=True)
    out_ref[4:5, :] += jnp.sum(s, axis=0, keepdims=True)


def kernel(adj, cur_sub_idx, uni_feat, sub_feat, original_sub_feat):
    sidx = jnp.asarray(cur_sub_idx, jnp.int32).reshape((1,))
    orig = original_sub_feat.reshape((1, D))
    grid_spec = pltpu.PrefetchScalarGridSpec(
        num_scalar_prefetch=1,
        grid=(GRID,),
        in_specs=[
            pl.BlockSpec((BLK, 128), lambda g, s: (g, s[0] // 128)),
            pl.BlockSpec((BLK, D), lambda g, s: (g, 0)),
            pl.BlockSpec((BLK, D), lambda g, s: (g, 0)),
            pl.BlockSpec((8, D), lambda g, s: (s[0] // 8, 0)),
            pl.BlockSpec((1, D), lambda g, s: (0, 0)),
        ],
        out_specs=pl.BlockSpec((8, D), lambda g, s: (0, 0)),
    )
    out = pl.pallas_call(
        _body,
        grid_spec=grid_spec,
        out_shape=jax.ShapeDtypeStruct((8, D), jnp.float32),
    )(sidx, adj, uni_feat, sub_feat, sub_feat, orig)
    return out[:5].reshape(-1)
